# baseline (speedup 1.0000x reference)
import functools

import jax
import jax.numpy as jnp
from jax import lax
from jax.experimental import pallas as pl
from jax.experimental.pallas import tpu as pltpu
from jax.experimental.pallas import tpu_sc as plsc

_NUM_CLASSES = 100000
_SPECIAL_ROW = _NUM_CLASSES + 1
_N = 16384
_D = 64
_PACKED_ROWS = 50001
_IDX_CHUNK = 128
_LANES = 16


@functools.lru_cache(maxsize=None)
def _make_lookup():
    info = plsc.get_sparse_core_info()
    nw = info.num_cores * info.num_subcores
    bpw = _N // nw
    n_chunks = bpw // _IDX_CHUNK
    mesh = plsc.VectorSubcoreMesh(core_axis_name="c", subcore_axis_name="s")

    @functools.partial(
        pl.kernel,
        mesh=mesh,
        out_type=jax.ShapeDtypeStruct((_D, _N), jnp.float32),
        scratch_types=[
            pltpu.VMEM((bpw,), jnp.int32),
            pltpu.VMEM((bpw,), jnp.int32),
            pltpu.VMEM((bpw, 128), jnp.float32),
            pltpu.VMEM((_D, bpw), jnp.float32),
            pltpu.SemaphoreType.DMA,
        ],
        compiler_params=pltpu.CompilerParams(
            use_tc_tiling_on_sc=True, needs_layout_passes=False
        ),
    )
    def lookup(labels_hbm, packed_hbm, out_hbm, idx_v, par_v, rows_v, slab_v,
               sem):
        wid = lax.axis_index("s") * info.num_cores + lax.axis_index("c")
        base = wid * bpw
        pltpu.sync_copy(labels_hbm.at[pl.ds(base, bpw)], idx_v)
        for i in range(bpw // _LANES):
            sl = pl.ds(i * _LANES, _LANES)
            v = idx_v[sl]
            v = jnp.where(v < 0, _SPECIAL_ROW, v)
            idx_v[sl] = v >> 1
            par_v[sl] = (v & 1) * _D
        copies = [
            pltpu.async_copy(
                packed_hbm.at[idx_v.at[pl.ds(j * _IDX_CHUNK, _IDX_CHUNK)]],
                rows_v.at[pl.ds(j * _IDX_CHUNK, _IDX_CHUNK)],
                sem,
            )
            for j in range(n_chunks)
        ]
        for c in copies:
            c.wait()
        for k in range(bpw // _LANES):
            sl = pl.ds(k * _LANES, _LANES)
            row_idx = lax.iota(jnp.int32, _LANES) + (k * _LANES)
            col0 = par_v[sl]
            for c in range(_D):
                slab_v[c, sl] = plsc.load_gather(rows_v, [row_idx, col0 + c])
        pltpu.sync_copy(slab_v, out_hbm.at[:, pl.ds(base, bpw)])

    return lookup


def kernel(labels, train, embedding_table):
    if labels.ndim == 0:
        labels = labels[None]
    lookup = _make_lookup()
    packed = jnp.reshape(embedding_table, (_PACKED_ROWS, 128))
    out_t = lookup(labels, packed)
    return out_t.T


# speedup vs baseline: 1.3518x; 1.3518x over previous
import functools

import jax
import jax.numpy as jnp
from jax import lax
from jax.experimental import pallas as pl
from jax.experimental.pallas import tpu as pltpu
from jax.experimental.pallas import tpu_sc as plsc

_NUM_CLASSES = 100000
_SPECIAL_ROW = _NUM_CLASSES + 1
_N = 16384
_D = 64
_LANES = 16
_WAVE = 32


@functools.lru_cache(maxsize=None)
def _make_lookup():
    info = plsc.get_sparse_core_info()
    nw = info.num_cores * info.num_subcores
    bpw = _N // nw
    n_waves = bpw // _WAVE
    mesh = plsc.VectorSubcoreMesh(core_axis_name="c", subcore_axis_name="s")

    @functools.partial(
        pl.kernel,
        mesh=mesh,
        out_type=jax.ShapeDtypeStruct((_N, _D), jnp.float32),
        scratch_types=[
            pltpu.VMEM((bpw,), jnp.int32),
            pltpu.VMEM((2, _WAVE, 8, _D), jnp.float32),
            pltpu.VMEM((_WAVE, _D), jnp.float32),
            pltpu.SemaphoreType.DMA,
            pltpu.SemaphoreType.DMA,
        ],
        compiler_params=pltpu.CompilerParams(use_tc_tiling_on_sc=True),
    )
    def lookup(labels_hbm, table_hbm, out_hbm, idx_v, tiles_v, wout_v, sem0,
               sem1):
        wid = lax.axis_index("s") * info.num_cores + lax.axis_index("c")
        base = wid * bpw
        sems = (sem0, sem1)
        pltpu.sync_copy(labels_hbm.at[pl.ds(base, bpw)], idx_v)
        for i in range(bpw // _LANES):
            sl = pl.ds(i * _LANES, _LANES)
            v = idx_v[sl]
            idx_v[sl] = jnp.where(v < 0, _SPECIAL_ROW, v)

        def fire(w, b):
            w0 = w * _WAVE
            for chunk in range(_WAVE // _LANES):
                v = idx_v[pl.ds(w0 + chunk * _LANES, _LANES)]
                for t in range(_LANES):
                    r = v[t]
                    rb = pl.multiple_of((r >> 3) * 8, 8)
                    pltpu.async_copy(
                        table_hbm.at[pl.ds(rb, 8), :],
                        tiles_v.at[b, chunk * _LANES + t],
                        sems[b],
                    )

        def consume(w, b):
            w0 = w * _WAVE
            for chunk in range(_WAVE // _LANES):
                v = idx_v[pl.ds(w0 + chunk * _LANES, _LANES)]
                for t in range(_LANES):
                    pltpu.make_async_copy(
                        table_hbm.at[pl.ds(0, 8), :],
                        tiles_v.at[b, chunk * _LANES + t],
                        sems[b],
                    ).wait()
                    rlow = v[t] & 7
                    for c in range(_D // _LANES):
                        sl = pl.ds(c * _LANES, _LANES)
                        wout_v[chunk * _LANES + t, sl] = tiles_v[
                            b, chunk * _LANES + t, rlow, sl
                        ]
            pltpu.sync_copy(wout_v, out_hbm.at[pl.ds(base + w0, _WAVE)])

        fire(0, 0)

        def pair(w2, _):
            w = 2 * w2
            fire(w + 1, 1)
            consume(w, 0)

            @pl.when(w + 2 < n_waves)
            def _():
                fire(w + 2, 0)

            consume(w + 1, 1)
            return 0

        lax.fori_loop(0, n_waves // 2, pair, 0)

    return lookup


def kernel(labels, train, embedding_table):
    if labels.ndim == 0:
        labels = labels[None]
    lookup = _make_lookup()
    return lookup(labels, embedding_table)
